# Initial kernel scaffold; baseline (speedup 1.0000x reference)
import jax
import jax.numpy as jnp
from jax.experimental import pallas as pl


def kernel(outputs, targets, mask):
    raise NotImplementedError("write your pallas kernel here")


# speedup vs baseline: 95.6411x; 95.6411x over previous
import jax
import jax.numpy as jnp
from jax.experimental import pallas as pl
from jax.experimental.pallas import tpu as pltpu

_B, _H, _W = 32, 512, 512
_NC = 10
_LANES = 128
_ROWS = (_H * _W) // _LANES
_CORES = 2
_STEPS = _B // _CORES


def _partial_kernel(o_ref, t_ref, m_ref, ps_ref, pc_ref):
    j = pl.program_id(1)
    o = o_ref[0]
    t = t_ref[0]
    mf = (m_ref[0] == 1).astype(jnp.float32)
    d = o - t.astype(jnp.float32)
    d2 = d * d * mf

    srows = []
    crows = []
    for c in range(_NC):
        eq = t == c
        srows.append(jnp.sum(jnp.where(eq, d2, 0.0), axis=0, keepdims=True))
        crows.append(jnp.sum(jnp.where(eq, mf, 0.0), axis=0, keepdims=True))
    pad = jnp.zeros((16 - _NC, _LANES), jnp.float32)
    s16 = jnp.concatenate(srows + [pad], axis=0)
    c16 = jnp.concatenate(crows + [pad], axis=0)

    @pl.when(j == 0)
    def _():
        ps_ref[0] = s16
        pc_ref[0] = c16

    @pl.when(j > 0)
    def _():
        ps_ref[0] = ps_ref[0] + s16
        pc_ref[0] = pc_ref[0] + c16


def _final_kernel(ps_ref, pc_ref, loss_ref, l4_ref, cn_ref):
    ps = ps_ref[0] + ps_ref[1]
    pc = pc_ref[0] + pc_ref[1]
    s = jnp.sum(ps, axis=1, keepdims=True)
    n = jnp.sum(pc, axis=1, keepdims=True)
    l4 = jnp.where(n > 0, s / jnp.maximum(n, 1.0), 0.0)
    l4_b = jnp.broadcast_to(l4, (16, _LANES))
    n_b = jnp.broadcast_to(n, (16, _LANES))
    l4_ref[...] = l4_b
    cn_ref[...] = n_b
    loss_ref[...] = 0.1 * jnp.sum(l4_b, axis=0, keepdims=True)


def kernel(outputs, targets, mask):
    o3 = outputs.reshape(_B, _ROWS, _LANES)
    t3 = targets.reshape(_B, _ROWS, _LANES)
    m3 = mask.reshape(_B, _ROWS, _LANES)

    blk = pl.BlockSpec((1, _ROWS, _LANES), lambda i, j: (i * _STEPS + j, 0, 0))
    acc = pl.BlockSpec((1, 16, _LANES), lambda i, j: (i, 0, 0))

    ps, pc = pl.pallas_call(
        _partial_kernel,
        grid=(_CORES, _STEPS),
        in_specs=[blk, blk, blk],
        out_specs=[acc, acc],
        out_shape=[
            jax.ShapeDtypeStruct((_CORES, 16, _LANES), jnp.float32),
            jax.ShapeDtypeStruct((_CORES, 16, _LANES), jnp.float32),
        ],
        compiler_params=pltpu.CompilerParams(
            dimension_semantics=("parallel", "arbitrary"),
        ),
    )(o3, t3, m3)

    loss_m, l4_m, cn_m = pl.pallas_call(
        _final_kernel,
        out_shape=[
            jax.ShapeDtypeStruct((1, _LANES), jnp.float32),
            jax.ShapeDtypeStruct((16, _LANES), jnp.float32),
            jax.ShapeDtypeStruct((16, _LANES), jnp.float32),
        ],
    )(ps, pc)

    loss = loss_m[0, 0]
    loss4each = l4_m[:_NC, 0]
    class_n = cn_m[:_NC, 0]
    return loss, loss4each, class_n


# speedup vs baseline: 205.5198x; 2.1489x over previous
import jax
import jax.numpy as jnp
from jax.experimental import pallas as pl
from jax.experimental.pallas import tpu as pltpu

_B, _H, _W = 32, 512, 512
_NC = 10
_CORES = 2
_STEPS = _B // _CORES


def _partial_kernel(o_ref, t_ref, m_ref, ps_ref, pc_ref):
    j = pl.program_id(1)
    o = o_ref[0]
    t = t_ref[0]
    tm = jnp.where(m_ref[0] == 1, t, _NC)
    d = o - t.astype(jnp.float32)
    d2 = d * d

    srows = []
    crows = []
    for c in range(_NC):
        eq = tm == c
        srows.append(jnp.sum(jnp.where(eq, d2, 0.0), axis=0, keepdims=True))
        crows.append(jnp.sum(jnp.where(eq, 1.0, 0.0), axis=0, keepdims=True))
    pad = jnp.zeros((16 - _NC, _W), jnp.float32)
    s16 = jnp.concatenate(srows + [pad], axis=0)
    c16 = jnp.concatenate(crows + [pad], axis=0)

    @pl.when(j == 0)
    def _():
        ps_ref[0] = s16
        pc_ref[0] = c16

    @pl.when(j > 0)
    def _():
        ps_ref[0] = ps_ref[0] + s16
        pc_ref[0] = pc_ref[0] + c16


def _final_kernel(ps_ref, pc_ref, loss_ref, l4_ref, cn_ref):
    ps = ps_ref[0] + ps_ref[1]
    pc = pc_ref[0] + pc_ref[1]
    s = jnp.sum(ps, axis=1, keepdims=True)
    n = jnp.sum(pc, axis=1, keepdims=True)
    l4 = jnp.where(n > 0, s / jnp.maximum(n, 1.0), 0.0)
    l4_b = jnp.broadcast_to(l4, (16, 128))
    n_b = jnp.broadcast_to(n, (16, 128))
    l4_ref[...] = l4_b
    cn_ref[...] = n_b
    loss_ref[...] = 0.1 * jnp.sum(l4_b, axis=0, keepdims=True)


def kernel(outputs, targets, mask):
    blk = pl.BlockSpec((1, _H, _W), lambda i, j: (i * _STEPS + j, 0, 0))
    acc = pl.BlockSpec((1, 16, _W), lambda i, j: (i, 0, 0))

    ps, pc = pl.pallas_call(
        _partial_kernel,
        grid=(_CORES, _STEPS),
        in_specs=[blk, blk, blk],
        out_specs=[acc, acc],
        out_shape=[
            jax.ShapeDtypeStruct((_CORES, 16, _W), jnp.float32),
            jax.ShapeDtypeStruct((_CORES, 16, _W), jnp.float32),
        ],
        compiler_params=pltpu.CompilerParams(
            dimension_semantics=("parallel", "arbitrary"),
        ),
    )(outputs, targets, mask)

    loss_m, l4_m, cn_m = pl.pallas_call(
        _final_kernel,
        out_shape=[
            jax.ShapeDtypeStruct((1, 128), jnp.float32),
            jax.ShapeDtypeStruct((16, 128), jnp.float32),
            jax.ShapeDtypeStruct((16, 128), jnp.float32),
        ],
    )(ps, pc)

    loss = loss_m[0, 0]
    loss4each = l4_m[:_NC, 0]
    class_n = cn_m[:_NC, 0]
    return loss, loss4each, class_n


# speedup vs baseline: 223.7525x; 1.0887x over previous
import jax
import jax.numpy as jnp
from jax.experimental import pallas as pl
from jax.experimental.pallas import tpu as pltpu

_B, _H, _W = 32, 512, 512
_NC = 10
_CORES = 2
_STEPS = _B // _CORES


def _partial_kernel(o_ref, t_ref, m_ref, ps_ref, pc_ref):
    j = pl.program_id(1)
    o = o_ref[0]
    t = t_ref[0]
    tm = jnp.where(m_ref[0] == 1, t, _NC)
    d = o - t.astype(jnp.float32)
    d2 = d * d

    srows = []
    crows = []
    for c in range(_NC):
        ef = jnp.where(tm == c, 1.0, 0.0)
        srows.append(jnp.sum(ef * d2, axis=0, keepdims=True))
        crows.append(jnp.sum(ef, axis=0, keepdims=True))
    pad = jnp.zeros((16 - _NC, _W), jnp.float32)
    s16 = jnp.concatenate(srows + [pad], axis=0)
    c16 = jnp.concatenate(crows + [pad], axis=0)

    @pl.when(j == 0)
    def _():
        ps_ref[0] = s16
        pc_ref[0] = c16

    @pl.when(j > 0)
    def _():
        ps_ref[0] = ps_ref[0] + s16
        pc_ref[0] = pc_ref[0] + c16


def _final_kernel(ps_ref, pc_ref, loss_ref, l4_ref, cn_ref):
    ps = ps_ref[0] + ps_ref[1]
    pc = pc_ref[0] + pc_ref[1]
    s = jnp.sum(ps, axis=1, keepdims=True)
    n = jnp.sum(pc, axis=1, keepdims=True)
    l4 = jnp.where(n > 0, s / jnp.maximum(n, 1.0), 0.0)
    l4_b = jnp.broadcast_to(l4, (16, 128))
    n_b = jnp.broadcast_to(n, (16, 128))
    l4_ref[...] = l4_b
    cn_ref[...] = n_b
    loss_ref[...] = 0.1 * jnp.sum(l4_b, axis=0, keepdims=True)


def kernel(outputs, targets, mask):
    blk = pl.BlockSpec((1, _H, _W), lambda i, j: (i * _STEPS + j, 0, 0))
    acc = pl.BlockSpec((1, 16, _W), lambda i, j: (i, 0, 0))

    ps, pc = pl.pallas_call(
        _partial_kernel,
        grid=(_CORES, _STEPS),
        in_specs=[blk, blk, blk],
        out_specs=[acc, acc],
        out_shape=[
            jax.ShapeDtypeStruct((_CORES, 16, _W), jnp.float32),
            jax.ShapeDtypeStruct((_CORES, 16, _W), jnp.float32),
        ],
        compiler_params=pltpu.CompilerParams(
            dimension_semantics=("parallel", "arbitrary"),
        ),
    )(outputs, targets, mask)

    loss_m, l4_m, cn_m = pl.pallas_call(
        _final_kernel,
        out_shape=[
            jax.ShapeDtypeStruct((1, 128), jnp.float32),
            jax.ShapeDtypeStruct((16, 128), jnp.float32),
            jax.ShapeDtypeStruct((16, 128), jnp.float32),
        ],
    )(ps, pc)

    loss = loss_m[0, 0]
    loss4each = l4_m[:_NC, 0]
    class_n = cn_m[:_NC, 0]
    return loss, loss4each, class_n


# speedup vs baseline: 224.5168x; 1.0034x over previous
import jax
import jax.numpy as jnp
from jax.experimental import pallas as pl
from jax.experimental.pallas import tpu as pltpu

_B, _H, _W = 32, 512, 512
_NC = 10
_BB = 2
_STEPS = _B // _BB


def _kernel(o_ref, t_ref, m_ref, loss_ref, l4_ref, cn_ref, ps, pc):
    j = pl.program_id(0)
    o = o_ref[...].reshape(_BB * _H, _W)
    t = t_ref[...].reshape(_BB * _H, _W)
    m = m_ref[...].reshape(_BB * _H, _W)
    tm = jnp.where(m == 1, t, _NC)
    d = o - t.astype(jnp.float32)
    d2 = d * d

    srows = []
    crows = []
    for c in range(_NC):
        ef = jnp.where(tm == c, 1.0, 0.0)
        srows.append(jnp.sum(ef * d2, axis=0, keepdims=True))
        crows.append(jnp.sum(ef, axis=0, keepdims=True))
    pad = jnp.zeros((16 - _NC, _W), jnp.float32)
    s16 = jnp.concatenate(srows + [pad], axis=0)
    c16 = jnp.concatenate(crows + [pad], axis=0)

    @pl.when(j == 0)
    def _():
        ps[...] = s16
        pc[...] = c16

    @pl.when(j > 0)
    def _():
        ps[...] = ps[...] + s16
        pc[...] = pc[...] + c16

    @pl.when(j == _STEPS - 1)
    def _():
        s = jnp.sum(ps[...], axis=1, keepdims=True)
        n = jnp.sum(pc[...], axis=1, keepdims=True)
        l4 = jnp.where(n > 0, s / jnp.maximum(n, 1.0), 0.0)
        l4_b = jnp.broadcast_to(l4, (16, 128))
        n_b = jnp.broadcast_to(n, (16, 128))
        l4_ref[...] = l4_b
        cn_ref[...] = n_b
        loss_ref[...] = 0.1 * jnp.sum(l4_b, axis=0, keepdims=True)


def kernel(outputs, targets, mask):
    blk = pl.BlockSpec((_BB, _H, _W), lambda j: (j, 0, 0))
    out = pl.BlockSpec((1, 128), lambda j: (0, 0))
    out16 = pl.BlockSpec((16, 128), lambda j: (0, 0))

    loss_m, l4_m, cn_m = pl.pallas_call(
        _kernel,
        grid=(_STEPS,),
        in_specs=[blk, blk, blk],
        out_specs=[out, out16, out16],
        out_shape=[
            jax.ShapeDtypeStruct((1, 128), jnp.float32),
            jax.ShapeDtypeStruct((16, 128), jnp.float32),
            jax.ShapeDtypeStruct((16, 128), jnp.float32),
        ],
        scratch_shapes=[
            pltpu.VMEM((16, _W), jnp.float32),
            pltpu.VMEM((16, _W), jnp.float32),
        ],
        compiler_params=pltpu.CompilerParams(
            dimension_semantics=("arbitrary",),
        ),
    )(outputs, targets, mask)

    loss = loss_m[0, 0]
    loss4each = l4_m[:_NC, 0]
    class_n = cn_m[:_NC, 0]
    return loss, loss4each, class_n


# speedup vs baseline: 273.1541x; 1.2166x over previous
import jax
import jax.numpy as jnp
from jax.experimental import pallas as pl
from jax.experimental.pallas import tpu as pltpu

_B, _H, _W = 32, 512, 512
_NC = 10
_BB = 2
_STEPS = _B // _BB
_CH = 64


def _kernel(o_ref, t_ref, m_ref, loss_ref, l4_ref, cn_ref, psA, pcA):
    j = pl.program_id(0)

    @pl.when(j == 0)
    def _():
        psA[...] = jnp.zeros((16, 8, _W), jnp.float32)
        pcA[...] = jnp.zeros((16, 8, _W), jnp.float32)

    for b in range(_BB):
        for hg in range(_H // _CH):
            oc = o_ref[b, hg * _CH:(hg + 1) * _CH, :]
            tc = t_ref[b, hg * _CH:(hg + 1) * _CH, :]
            mc = m_ref[b, hg * _CH:(hg + 1) * _CH, :]
            tmc = jnp.where(mc == 1, tc, _NC)
            dd = oc - tc.astype(jnp.float32)
            d2c = dd * dd
            for c in range(_NC):
                ef = jnp.where(tmc == c, 1.0, 0.0)
                psA[c] += jnp.sum((ef * d2c).reshape(_CH // 8, 8, _W), axis=0)
                pcA[c] += jnp.sum(ef.reshape(_CH // 8, 8, _W), axis=0)

    @pl.when(j == _STEPS - 1)
    def _():
        ps16 = jnp.sum(psA[...], axis=1)
        pc16 = jnp.sum(pcA[...], axis=1)
        s = jnp.sum(ps16, axis=1, keepdims=True)
        n = jnp.sum(pc16, axis=1, keepdims=True)
        l4 = jnp.where(n > 0, s / jnp.maximum(n, 1.0), 0.0)
        l4_b = jnp.broadcast_to(l4, (16, 128))
        n_b = jnp.broadcast_to(n, (16, 128))
        l4_ref[...] = l4_b
        cn_ref[...] = n_b
        loss_ref[...] = 0.1 * jnp.sum(l4_b, axis=0, keepdims=True)


def kernel(outputs, targets, mask):
    blk = pl.BlockSpec((_BB, _H, _W), lambda j: (j, 0, 0))
    out = pl.BlockSpec((1, 128), lambda j: (0, 0))
    out16 = pl.BlockSpec((16, 128), lambda j: (0, 0))

    loss_m, l4_m, cn_m = pl.pallas_call(
        _kernel,
        grid=(_STEPS,),
        in_specs=[blk, blk, blk],
        out_specs=[out, out16, out16],
        out_shape=[
            jax.ShapeDtypeStruct((1, 128), jnp.float32),
            jax.ShapeDtypeStruct((16, 128), jnp.float32),
            jax.ShapeDtypeStruct((16, 128), jnp.float32),
        ],
        scratch_shapes=[
            pltpu.VMEM((16, 8, _W), jnp.float32),
            pltpu.VMEM((16, 8, _W), jnp.float32),
        ],
        compiler_params=pltpu.CompilerParams(
            dimension_semantics=("arbitrary",),
        ),
    )(outputs, targets, mask)

    loss = loss_m[0, 0]
    loss4each = l4_m[:_NC, 0]
    class_n = cn_m[:_NC, 0]
    return loss, loss4each, class_n
